# Initial kernel scaffold; baseline (speedup 1.0000x reference)
import jax
import jax.numpy as jnp
from jax.experimental import pallas as pl


def kernel(x, edge_index, edge_types, W_o, b_o, W_i, b_i, W_s, b_s):
    raise NotImplementedError("write your pallas kernel here")


# speedup vs baseline: 2.2075x; 2.2075x over previous
import functools

import jax
import jax.numpy as jnp
from jax import lax
from jax.experimental import pallas as pl
from jax.experimental.pallas import tpu as pltpu
from jax.experimental.pallas import tpu_sc as plsc

N = 10000
E = 320000
D = 128
R = 8

NC = 2
NS = 16
H = N // NC
EPT = E // NS

AROWS = 15104
AZ = AROWS // NS
CROWS = 5120
CZ = CROWS // NS
CW = 128

CA = 32
MA = 480
NBLK = EPT // MA
TAIL = EPT - NBLK * MA
CC = 80
MC = 400

BN = 200
NB = H // BN


def _cnt_body(ei_hbm, et_hbm, c_out,
              dst_v, typ_v, cidx0_v, cidx1_v, cidx2_v, oh0_v, oh1_v, oh2_v,
              c_sp):
    c_id = lax.axis_index("c")
    s_id = lax.axis_index("s")
    base = c_id * H

    zf = jnp.zeros((16,), jnp.float32)
    lane = lax.iota(jnp.int32, 16)
    one = jnp.broadcast_to(jnp.float32(1.0), (16,))
    zero = jnp.broadcast_to(jnp.float32(0.0), (16,))
    rows = [jnp.where(lane == f, one, zero) for f in range(3)]
    ohs = [oh0_v, oh1_v, oh2_v]

    def _zzero(i, _):
        for j in range(CW // 16):
            oh0_v[i, pl.ds(j * 16, 16)] = zf
        return 0
    lax.fori_loop(0, CC, _zzero, 0)
    for j in range(CZ // CC):
        pltpu.sync_copy(oh0_v, c_sp.at[pl.ds(s_id * CZ + j * CC, CC)])

    def _zrow(i, _):
        for f in range(3):
            ohs[f][i, pl.ds(0, 16)] = rows[f]
            for j in range(1, CW // 16):
                ohs[f][i, pl.ds(j * 16, 16)] = zf
        return 0
    lax.fori_loop(0, CC, _zrow, 0)
    plsc.subcore_barrier()

    tbase = s_id * EPT
    base_v = jnp.broadcast_to(base, (16,))

    def _block(k, _):
        moff = tbase + k * MC
        pltpu.sync_copy(ei_hbm.at[pl.ds(E + moff, MC)], dst_v)
        pltpu.sync_copy(et_hbm.at[pl.ds(moff, MC)], typ_v)
        for j in range(MC // CC):
            for g in range(CC // 16):
                sl = pl.ds(g * 16, 16)
                t = typ_v[pl.ds(j * CC + g * 16, 16)]
                d = dst_v[pl.ds(j * CC + g * 16, 16)]
                inr = (d >= base_v) & (d < base_v + H)
                rel = d - base_v
                m0 = inr & (t < R)
                m1 = inr & (t >= R) & (t < 2 * R)
                m2 = inr & (t >= 2 * R)
                tr = (s_id + g * NS) % 40
                cidx0_v[sl] = jnp.where(
                    m0, rel, jnp.broadcast_to(H + tr, (16,)))
                cidx1_v[sl] = jnp.where(
                    m1, rel, jnp.broadcast_to(H + 40 + tr, (16,)))
                cidx2_v[sl] = jnp.where(
                    m2, rel, jnp.broadcast_to(H + 80 + tr, (16,)))
            pltpu.sync_copy(oh0_v, c_sp.at[cidx0_v], add=True)
            pltpu.sync_copy(oh1_v, c_sp.at[cidx1_v], add=True)
            pltpu.sync_copy(oh2_v, c_sp.at[cidx2_v], add=True)
        return 0
    lax.fori_loop(0, EPT // MC, _block, 0)
    plsc.subcore_barrier()

    pltpu.sync_copy(c_sp.at[pl.ds(s_id * CZ, CZ)],
                    c_out.at[c_id, pl.ds(s_id * CZ, CZ)])


def _acc_body(x_hbm, ei_hbm, et_hbm, cnt_hbm, a_out,
              src_v, dst_v, typ_v, fidx_v, rows_v, a_sp):
    del cnt_hbm
    c_id = lax.axis_index("c")
    s_id = lax.axis_index("s")
    base = c_id * H

    zf = jnp.zeros((16,), jnp.float32)

    def _zrow(i, _):
        for j in range(D // 16):
            rows_v[i, pl.ds(j * 16, 16)] = zf
        return 0
    lax.fori_loop(0, CA, _zrow, 0)
    for j in range(AZ // CA):
        pltpu.sync_copy(rows_v, a_sp.at[pl.ds(s_id * AZ + j * CA, CA)])
    rem = AZ % CA
    if rem:
        pltpu.sync_copy(rows_v.at[pl.ds(0, rem)],
                        a_sp.at[pl.ds(s_id * AZ + (AZ // CA) * CA, rem)])
    plsc.subcore_barrier()

    tbase = s_id * EPT
    base_v = jnp.broadcast_to(base, (16,))
    fam0 = jnp.broadcast_to(jnp.int32(0), (16,))
    famH = jnp.broadcast_to(jnp.int32(H), (16,))
    fam2H = jnp.broadcast_to(jnp.int32(2 * H), (16,))

    def _chunk(j):
        for g in range(CA // 16):
            sl = pl.ds(g * 16, 16)
            t = typ_v[pl.ds(j * CA + g * 16, 16)]
            d = dst_v[pl.ds(j * CA + g * 16, 16)]
            inr = (d >= base_v) & (d < base_v + H)
            fbase = jnp.where(t < R, fam0, jnp.where(t < 2 * R, famH, fam2H))
            trash = jnp.broadcast_to(
                3 * H + (s_id + g * NS) % (AROWS - 3 * H), (16,))
            fidx_v[sl] = jnp.where(inr, fbase + (d - base_v), trash)
        pltpu.sync_copy(x_hbm.at[src_v.at[pl.ds(j * CA, CA)]], rows_v)
        pltpu.sync_copy(rows_v, a_sp.at[fidx_v], add=True)

    def _block(k, _):
        moff = tbase + k * MA
        pltpu.sync_copy(ei_hbm.at[pl.ds(moff, MA)], src_v)
        pltpu.sync_copy(ei_hbm.at[pl.ds(E + moff, MA)], dst_v)
        pltpu.sync_copy(et_hbm.at[pl.ds(moff, MA)], typ_v)
        for j in range(MA // CA):
            _chunk(j)
        return 0
    lax.fori_loop(0, NBLK, _block, 0)

    moff = tbase + NBLK * MA
    pltpu.sync_copy(ei_hbm.at[pl.ds(moff, TAIL)], src_v.at[pl.ds(0, TAIL)])
    pltpu.sync_copy(ei_hbm.at[pl.ds(E + moff, TAIL)], dst_v.at[pl.ds(0, TAIL)])
    pltpu.sync_copy(et_hbm.at[pl.ds(moff, TAIL)], typ_v.at[pl.ds(0, TAIL)])
    for j in range(TAIL // CA):
        _chunk(j)
    plsc.subcore_barrier()

    pltpu.sync_copy(a_sp.at[pl.ds(s_id * AZ, AZ)],
                    a_out.at[c_id, pl.ds(s_id * AZ, AZ)])


def _make_sc_calls():
    mesh = plsc.VectorSubcoreMesh(core_axis_name="c", subcore_axis_name="s",
                                  num_cores=NC, num_subcores=NS)
    cnt_call = pl.kernel(
        _cnt_body,
        out_type=jax.ShapeDtypeStruct((NC, CROWS, CW), jnp.float32),
        mesh=mesh,
        scratch_types=[
            pltpu.VMEM((MC,), jnp.int32),
            pltpu.VMEM((MC,), jnp.int32),
            pltpu.VMEM((CC,), jnp.int32),
            pltpu.VMEM((CC,), jnp.int32),
            pltpu.VMEM((CC,), jnp.int32),
            pltpu.VMEM((CC, CW), jnp.float32),
            pltpu.VMEM((CC, CW), jnp.float32),
            pltpu.VMEM((CC, CW), jnp.float32),
            pltpu.VMEM_SHARED((CROWS, CW), jnp.float32),
        ],
    )
    acc_call = pl.kernel(
        _acc_body,
        out_type=jax.ShapeDtypeStruct((NC, AROWS, D), jnp.float32),
        mesh=mesh,
        scratch_types=[
            pltpu.VMEM((MA,), jnp.int32),
            pltpu.VMEM((MA,), jnp.int32),
            pltpu.VMEM((MA,), jnp.int32),
            pltpu.VMEM((CA,), jnp.int32),
            pltpu.VMEM((CA, D), jnp.float32),
            pltpu.VMEM_SHARED((AROWS, D), jnp.float32),
        ],
    )
    return cnt_call, acc_call


_cnt_call, _acc_call = _make_sc_calls()


def _tc_body(a0_ref, a1_ref, a2_ref, cnt_ref, wo_ref, wi_ref, ws_ref,
             bmat_ref, out_ref):
    dn = (((1,), (1,)), ((), ()))
    m = lax.dot_general(a0_ref[0], wo_ref[...], dn,
                        preferred_element_type=jnp.float32,
                        precision=lax.Precision.HIGHEST)
    m += lax.dot_general(a1_ref[0], wi_ref[...], dn,
                         preferred_element_type=jnp.float32,
                         precision=lax.Precision.HIGHEST)
    m += lax.dot_general(a2_ref[0], ws_ref[...], dn,
                         preferred_element_type=jnp.float32,
                         precision=lax.Precision.HIGHEST)
    cnt = cnt_ref[0]
    m += lax.dot_general(cnt, bmat_ref[...], (((1,), (0,)), ((), ())),
                         preferred_element_type=jnp.float32,
                         precision=lax.Precision.HIGHEST)
    ctot = jnp.sum(cnt, axis=1, keepdims=True)
    inv = 1.0 / jnp.maximum(ctot, 1.0)
    out_ref[...] = jnp.maximum(m * inv, 0.0)


_tc_call = pl.pallas_call(
    _tc_body,
    grid=(NC, NB),
    in_specs=[
        pl.BlockSpec((1, BN, D), lambda c, b: (c, 0 * NB + b, 0)),
        pl.BlockSpec((1, BN, D), lambda c, b: (c, 1 * NB + b, 0)),
        pl.BlockSpec((1, BN, D), lambda c, b: (c, 2 * NB + b, 0)),
        pl.BlockSpec((1, BN, CW), lambda c, b: (c, b, 0)),
        pl.BlockSpec((D, D), lambda c, b: (0, 0)),
        pl.BlockSpec((D, D), lambda c, b: (0, 0)),
        pl.BlockSpec((D, D), lambda c, b: (0, 0)),
        pl.BlockSpec((CW, D), lambda c, b: (0, 0)),
    ],
    out_specs=pl.BlockSpec((BN, D), lambda c, b: (c * NB + b, 0)),
    out_shape=jax.ShapeDtypeStruct((N, D), jnp.float32),
)


@jax.jit
def kernel(x, edge_index, edge_types, W_o, b_o, W_i, b_i, W_s, b_s):
    ei = edge_index.reshape(2 * E)
    cnt = _cnt_call(ei, edge_types)
    a_acc = _acc_call(x, ei, edge_types, cnt)
    bmat = jnp.zeros((CW, D), jnp.float32)
    bmat = bmat.at[0].set(b_o).at[1].set(b_i).at[2].set(b_s)
    return _tc_call(a_acc, a_acc, a_acc, cnt, W_o, W_i, W_s, bmat)


# speedup vs baseline: 3.5152x; 1.5924x over previous
import functools

import jax
import jax.numpy as jnp
from jax import lax
from jax.experimental import pallas as pl
from jax.experimental.pallas import tpu as pltpu
from jax.experimental.pallas import tpu_sc as plsc

N = 10000
E = 320000
D = 128
R = 8

NC = 2
NS = 16
H = N // NC
EPT = E // NS

AROWS = 15104
AZ = AROWS // NS
TRASH = AROWS - 3 * H

CA = 16
MA = 160
NBLK = EPT // MA
CPB = MA // CA
CC = 32
MC = 160
NBLKC = EPT // MC
CPBC = MC // CC

BN = 200
NB = H // BN


def _fam_base(t, d, base_v, s_id, g):
    inr = (d >= base_v) & (d < base_v + H)
    fam0 = jnp.broadcast_to(jnp.int32(0), (16,))
    famH = jnp.broadcast_to(jnp.int32(H), (16,))
    fam2H = jnp.broadcast_to(jnp.int32(2 * H), (16,))
    fbase = jnp.where(t < R, fam0, jnp.where(t < 2 * R, famH, fam2H))
    trash = jnp.broadcast_to(3 * H + (s_id + g * NS) % TRASH, (16,))
    return jnp.where(inr, fbase + (d - base_v), trash)


def _cnt_body(ei_hbm, et_hbm, c_out,
              dst_a, dst_b, typ_a, typ_b, cidx_a, cidx_b, oh_v, c_sp,
              msem_a, msem_b, ssem_a, ssem_b, zsem):
    c_id = lax.axis_index("c")
    s_id = lax.axis_index("s")
    base = c_id * H
    tbase = s_id * EPT
    base_v = jnp.broadcast_to(base, (16,))
    zf = jnp.zeros((16,), jnp.float32)

    def _zrow(i, _):
        for j in range(D // 16):
            oh_v[i, pl.ds(j * 16, 16)] = zf
        return 0
    lax.fori_loop(0, CC, _zrow, 0)
    nz = AZ // CC
    rz = AZ - nz * CC
    for j in range(nz):
        pltpu.async_copy(oh_v, c_sp.at[pl.ds(s_id * AZ + j * CC, CC)], zsem)
    pltpu.async_copy(oh_v.at[pl.ds(0, rz)],
                     c_sp.at[pl.ds(s_id * AZ + nz * CC, rz)], zsem)
    for j in range(nz):
        pltpu.make_async_copy(
            oh_v, c_sp.at[pl.ds(s_id * AZ + j * CC, CC)], zsem).wait()
    pltpu.make_async_copy(
        oh_v.at[pl.ds(0, rz)], c_sp.at[pl.ds(s_id * AZ + nz * CC, rz)],
        zsem).wait()
    lane = lax.iota(jnp.int32, 16)
    one = jnp.broadcast_to(jnp.float32(1.0), (16,))
    zero = jnp.broadcast_to(jnp.float32(0.0), (16,))
    oh_row = jnp.where(lane == 0, one, zero)

    def _orow(i, _):
        oh_v[i, pl.ds(0, 16)] = oh_row
        return 0
    lax.fori_loop(0, CC, _orow, 0)
    plsc.subcore_barrier()

    metas = ((dst_a, typ_a, msem_a), (dst_b, typ_b, msem_b))
    cidxs = (cidx_a, cidx_b)
    ssems = (ssem_a, ssem_b)

    def _fire_meta(bi, side):
        dst_x, typ_x, ms = metas[side]
        moff = tbase + bi * MC
        pltpu.async_copy(ei_hbm.at[pl.ds(E + moff, MC)], dst_x, ms)
        pltpu.async_copy(et_hbm.at[pl.ds(moff, MC)], typ_x, ms)

    def _wait_meta(side):
        dst_x, typ_x, ms = metas[side]
        pltpu.make_async_copy(ei_hbm.at[pl.ds(E, MC)], dst_x, ms).wait()
        pltpu.make_async_copy(et_hbm.at[pl.ds(0, MC)], typ_x, ms).wait()

    def _chunk(side, j, p, skip_wait):
        dst_x, typ_x, _ = metas[side]
        cx, sx = cidxs[p], ssems[p]
        if not skip_wait:
            pltpu.make_async_copy(oh_v, c_sp.at[cx], sx).wait()
        for g in range(CC // 16):
            sl = pl.ds(g * 16, 16)
            t = typ_x[pl.ds(j * CC + g * 16, 16)]
            d = dst_x[pl.ds(j * CC + g * 16, 16)]
            cx[sl] = _fam_base(t, d, base_v, s_id, g)
        pltpu.async_copy(oh_v, c_sp.at[cx], sx, add=True)

    def _ubody(bi, side, poff, fire_next):
        _wait_meta(side)
        if fire_next:
            _fire_meta(bi + 1, 1 - side)
        for j in range(CPBC):
            _chunk(side, j, (poff + j) % 2, False)

    _fire_meta(0, 0)
    _wait_meta(0)
    _fire_meta(1, 1)
    for j in range(CPBC):
        _chunk(0, j, j % 2, j < 2)
    _ubody(1, 1, 1, True)

    @pl.loop(2, NBLKC - 1, step=2)
    def _steady(b):
        _ubody(b, 0, 0, True)
        _ubody(b + 1, 1, 1, True)

    _ubody(NBLKC - 1, 0, 0, False)
    pltpu.make_async_copy(oh_v, c_sp.at[cidx_a], ssem_a).wait()
    pltpu.make_async_copy(oh_v, c_sp.at[cidx_b], ssem_b).wait()
    plsc.subcore_barrier()

    pltpu.sync_copy(c_sp.at[pl.ds(s_id * AZ, AZ)],
                    c_out.at[c_id, pl.ds(s_id * AZ, AZ)])


def _acc_body(x_hbm, ei_hbm, et_hbm, cnt_hbm, a_out,
              src_a, src_b, dst_a, dst_b, typ_a, typ_b,
              fidx_a, fidx_b, rows_a, rows_b, a_sp,
              msem_a, msem_b, gsem_a, gsem_b, ssem_a, ssem_b, zsem):
    del cnt_hbm
    c_id = lax.axis_index("c")
    s_id = lax.axis_index("s")
    base = c_id * H
    tbase = s_id * EPT
    base_v = jnp.broadcast_to(base, (16,))
    zf = jnp.zeros((16,), jnp.float32)

    def _zrow(i, _):
        for j in range(D // 16):
            rows_a[i, pl.ds(j * 16, 16)] = zf
            rows_b[i, pl.ds(j * 16, 16)] = zf
        return 0
    lax.fori_loop(0, CA, _zrow, 0)
    nz = AZ // (2 * CA)
    rz = AZ - nz * 2 * CA
    for j in range(nz):
        pltpu.async_copy(
            rows_a, a_sp.at[pl.ds(s_id * AZ + 2 * j * CA, CA)], zsem)
        pltpu.async_copy(
            rows_b, a_sp.at[pl.ds(s_id * AZ + (2 * j + 1) * CA, CA)], zsem)
    pltpu.async_copy(rows_a.at[pl.ds(0, rz)],
                     a_sp.at[pl.ds(s_id * AZ + nz * 2 * CA, rz)], zsem)
    for j in range(2 * nz):
        pltpu.make_async_copy(rows_a, a_sp.at[pl.ds(0, CA)], zsem).wait()
    pltpu.make_async_copy(rows_a.at[pl.ds(0, rz)],
                          a_sp.at[pl.ds(0, rz)], zsem).wait()
    plsc.subcore_barrier()

    metas = ((src_a, dst_a, typ_a, msem_a), (src_b, dst_b, typ_b, msem_b))
    fidxs = (fidx_a, fidx_b)
    rows = (rows_a, rows_b)
    gsems = (gsem_a, gsem_b)
    ssems = (ssem_a, ssem_b)

    def _fire_meta(bi, side):
        src_x, dst_x, typ_x, ms = metas[side]
        moff = tbase + bi * MA
        pltpu.async_copy(ei_hbm.at[pl.ds(moff, MA)], src_x, ms)
        pltpu.async_copy(ei_hbm.at[pl.ds(E + moff, MA)], dst_x, ms)
        pltpu.async_copy(et_hbm.at[pl.ds(moff, MA)], typ_x, ms)

    def _wait_meta(side):
        src_x, dst_x, typ_x, ms = metas[side]
        pltpu.make_async_copy(ei_hbm.at[pl.ds(0, MA)], src_x, ms).wait()
        pltpu.make_async_copy(ei_hbm.at[pl.ds(E, MA)], dst_x, ms).wait()
        pltpu.make_async_copy(et_hbm.at[pl.ds(0, MA)], typ_x, ms).wait()

    def _chunk(bi, side, j, skip_swait, skip_prev, fire_next):
        src_x, dst_x, typ_x, _ = metas[side]
        p = j % 2
        q = 1 - p
        if not skip_swait:
            pltpu.make_async_copy(rows[p], a_sp.at[fidxs[p]], ssems[p]).wait()
        g = 0
        t = typ_x[pl.ds(j * CA, 16)]
        d = dst_x[pl.ds(j * CA, 16)]
        fidxs[p][pl.ds(0, 16)] = _fam_base(t, d, base_v, s_id, g)
        pltpu.async_copy(x_hbm.at[src_x.at[pl.ds(j * CA, CA)]],
                         rows[p], gsems[p])
        if fire_next:
            _fire_meta(bi + 1, 1 - side)
        if not skip_prev:
            if j == 0:
                src_y = metas[1 - side][0]
                gsrc = x_hbm.at[src_y.at[pl.ds((CPB - 1) * CA, CA)]]
            else:
                gsrc = x_hbm.at[src_x.at[pl.ds((j - 1) * CA, CA)]]
            pltpu.make_async_copy(gsrc, rows[q], gsems[q]).wait()
            pltpu.async_copy(rows[q], a_sp.at[fidxs[q]], ssems[q], add=True)

    def _ubody(bi, side, fire_next):
        _wait_meta(side)
        for j in range(CPB):
            _chunk(bi, side, j, False, False, fire_next and j == 0)

    _fire_meta(0, 0)
    _wait_meta(0)
    for j in range(CPB):
        _chunk(0, 0, j, j < 2, j < 1, j == 0)
    _ubody(1, 1, True)

    @pl.loop(2, NBLK - 1, step=2)
    def _steady(b):
        _ubody(b, 0, True)
        _ubody(b + 1, 1, True)

    _ubody(NBLK - 1, 0, False)
    src_x = metas[0][0]
    pltpu.make_async_copy(x_hbm.at[src_x.at[pl.ds((CPB - 1) * CA, CA)]],
                          rows[1], gsems[1]).wait()
    pltpu.async_copy(rows[1], a_sp.at[fidxs[1]], ssems[1], add=True)
    pltpu.make_async_copy(rows[0], a_sp.at[fidxs[0]], ssems[0]).wait()
    pltpu.make_async_copy(rows[1], a_sp.at[fidxs[1]], ssems[1]).wait()
    plsc.subcore_barrier()

    pltpu.sync_copy(a_sp.at[pl.ds(s_id * AZ, AZ)],
                    a_out.at[c_id, pl.ds(s_id * AZ, AZ)])


def _make_sc_calls():
    mesh = plsc.VectorSubcoreMesh(core_axis_name="c", subcore_axis_name="s",
                                  num_cores=NC, num_subcores=NS)
    cnt_call = pl.kernel(
        _cnt_body,
        out_type=jax.ShapeDtypeStruct((NC, AROWS, D), jnp.float32),
        mesh=mesh,
        scratch_types=[
            pltpu.VMEM((MC,), jnp.int32),
            pltpu.VMEM((MC,), jnp.int32),
            pltpu.VMEM((MC,), jnp.int32),
            pltpu.VMEM((MC,), jnp.int32),
            pltpu.VMEM((CC,), jnp.int32),
            pltpu.VMEM((CC,), jnp.int32),
            pltpu.VMEM((CC, D), jnp.float32),
            pltpu.VMEM_SHARED((AROWS, D), jnp.float32),
            pltpu.SemaphoreType.DMA,
            pltpu.SemaphoreType.DMA,
            pltpu.SemaphoreType.DMA,
            pltpu.SemaphoreType.DMA,
            pltpu.SemaphoreType.DMA,
        ],
    )
    acc_call = pl.kernel(
        _acc_body,
        out_type=jax.ShapeDtypeStruct((NC, AROWS, D), jnp.float32),
        mesh=mesh,
        scratch_types=[
            pltpu.VMEM((MA,), jnp.int32),
            pltpu.VMEM((MA,), jnp.int32),
            pltpu.VMEM((MA,), jnp.int32),
            pltpu.VMEM((MA,), jnp.int32),
            pltpu.VMEM((MA,), jnp.int32),
            pltpu.VMEM((MA,), jnp.int32),
            pltpu.VMEM((CA,), jnp.int32),
            pltpu.VMEM((CA,), jnp.int32),
            pltpu.VMEM((CA, D), jnp.float32),
            pltpu.VMEM((CA, D), jnp.float32),
            pltpu.VMEM_SHARED((AROWS, D), jnp.float32),
            pltpu.SemaphoreType.DMA,
            pltpu.SemaphoreType.DMA,
            pltpu.SemaphoreType.DMA,
            pltpu.SemaphoreType.DMA,
            pltpu.SemaphoreType.DMA,
            pltpu.SemaphoreType.DMA,
            pltpu.SemaphoreType.DMA,
        ],
    )
    return cnt_call, acc_call


_cnt_call, _acc_call = _make_sc_calls()


def _tc_body(a0_ref, a1_ref, a2_ref, c0_ref, c1_ref, c2_ref,
             wo_ref, wi_ref, ws_ref, bo_ref, bi_ref, bs_ref, out_ref):
    dn = (((1,), (1,)), ((), ()))
    m = lax.dot_general(a0_ref[0], wo_ref[...], dn,
                        preferred_element_type=jnp.float32,
                        precision=lax.Precision.HIGHEST)
    m += lax.dot_general(a1_ref[0], wi_ref[...], dn,
                         preferred_element_type=jnp.float32,
                         precision=lax.Precision.HIGHEST)
    m += lax.dot_general(a2_ref[0], ws_ref[...], dn,
                         preferred_element_type=jnp.float32,
                         precision=lax.Precision.HIGHEST)
    c0 = c0_ref[0][:, 0:1]
    c1 = c1_ref[0][:, 0:1]
    c2 = c2_ref[0][:, 0:1]
    m += c0 * bo_ref[...] + c1 * bi_ref[...] + c2 * bs_ref[...]
    inv = 1.0 / jnp.maximum(c0 + c1 + c2, 1.0)
    out_ref[...] = jnp.maximum(m * inv, 0.0)


_tc_call = pl.pallas_call(
    _tc_body,
    grid=(NC, NB),
    in_specs=[
        pl.BlockSpec((1, BN, D), lambda c, b: (c, 0 * NB + b, 0)),
        pl.BlockSpec((1, BN, D), lambda c, b: (c, 1 * NB + b, 0)),
        pl.BlockSpec((1, BN, D), lambda c, b: (c, 2 * NB + b, 0)),
        pl.BlockSpec((1, BN, D), lambda c, b: (c, 0 * NB + b, 0)),
        pl.BlockSpec((1, BN, D), lambda c, b: (c, 1 * NB + b, 0)),
        pl.BlockSpec((1, BN, D), lambda c, b: (c, 2 * NB + b, 0)),
        pl.BlockSpec((D, D), lambda c, b: (0, 0)),
        pl.BlockSpec((D, D), lambda c, b: (0, 0)),
        pl.BlockSpec((D, D), lambda c, b: (0, 0)),
        pl.BlockSpec((1, D), lambda c, b: (0, 0)),
        pl.BlockSpec((1, D), lambda c, b: (0, 0)),
        pl.BlockSpec((1, D), lambda c, b: (0, 0)),
    ],
    out_specs=pl.BlockSpec((BN, D), lambda c, b: (c * NB + b, 0)),
    out_shape=jax.ShapeDtypeStruct((N, D), jnp.float32),
)


@jax.jit
def kernel(x, edge_index, edge_types, W_o, b_o, W_i, b_i, W_s, b_s):
    ei = edge_index.reshape(2 * E)
    cnt = _cnt_call(ei, edge_types)
    a_acc = _acc_call(x, ei, edge_types, cnt)
    return _tc_call(a_acc, a_acc, a_acc, cnt, cnt, cnt, W_o, W_i, W_s,
                    b_o.reshape(1, D), b_i.reshape(1, D), b_s.reshape(1, D))


# speedup vs baseline: 4.5180x; 1.2853x over previous
import functools

import jax
import jax.numpy as jnp
from jax import lax
from jax.experimental import pallas as pl
from jax.experimental.pallas import tpu as pltpu
from jax.experimental.pallas import tpu_sc as plsc

N = 10000
E = 320000
D = 128
R = 8

NC = 2
NS = 16
H = N // NC
EPT = E // NS

AROWS = 15104
AZ = AROWS // NS
TRASH = AROWS - 3 * H

CA = 16
MA = 240
NBLK = EPT // MA
CPB = MA // CA
TAILE = EPT - NBLK * MA
TAILC = TAILE // CA
CC = 32
MC = 160
NBLKC = EPT // MC
CPBC = MC // CC

BN = 200
NB = H // BN


def _fam_base(t, d, base_v, s_id, g):
    inr = (d >= base_v) & (d < base_v + H)
    fam0 = jnp.broadcast_to(jnp.int32(0), (16,))
    famH = jnp.broadcast_to(jnp.int32(H), (16,))
    fam2H = jnp.broadcast_to(jnp.int32(2 * H), (16,))
    fbase = jnp.where(t < R, fam0, jnp.where(t < 2 * R, famH, fam2H))
    trash = jnp.broadcast_to(3 * H + (s_id + g * NS) % TRASH, (16,))
    return jnp.where(inr, fbase + (d - base_v), trash)


def _cnt_body(ei_hbm, et_hbm, c_out,
              dst_a, dst_b, typ_a, typ_b, cidx_a, cidx_b, oh_v, c_sp,
              msem_a, msem_b, ssem_a, ssem_b, zsem):
    c_id = lax.axis_index("c")
    s_id = lax.axis_index("s")
    base = c_id * H
    tbase = s_id * EPT
    base_v = jnp.broadcast_to(base, (16,))
    zf = jnp.zeros((16,), jnp.float32)

    def _zrow(i, _):
        for j in range(D // 16):
            oh_v[i, pl.ds(j * 16, 16)] = zf
        return 0
    lax.fori_loop(0, CC, _zrow, 0)
    nz = AZ // CC
    rz = AZ - nz * CC
    for j in range(nz):
        pltpu.async_copy(oh_v, c_sp.at[pl.ds(s_id * AZ + j * CC, CC)], zsem)
    pltpu.async_copy(oh_v.at[pl.ds(0, rz)],
                     c_sp.at[pl.ds(s_id * AZ + nz * CC, rz)], zsem)
    for j in range(nz):
        pltpu.make_async_copy(
            oh_v, c_sp.at[pl.ds(s_id * AZ + j * CC, CC)], zsem).wait()
    pltpu.make_async_copy(
        oh_v.at[pl.ds(0, rz)], c_sp.at[pl.ds(s_id * AZ + nz * CC, rz)],
        zsem).wait()
    lane = lax.iota(jnp.int32, 16)
    one = jnp.broadcast_to(jnp.float32(1.0), (16,))
    zero = jnp.broadcast_to(jnp.float32(0.0), (16,))
    oh_row = jnp.where(lane == 0, one, zero)

    def _orow(i, _):
        oh_v[i, pl.ds(0, 16)] = oh_row
        return 0
    lax.fori_loop(0, CC, _orow, 0)
    plsc.subcore_barrier()

    metas = ((dst_a, typ_a, msem_a), (dst_b, typ_b, msem_b))
    cidxs = (cidx_a, cidx_b)
    ssems = (ssem_a, ssem_b)

    def _fire_meta(bi, side):
        dst_x, typ_x, ms = metas[side]
        moff = tbase + bi * MC
        pltpu.async_copy(ei_hbm.at[pl.ds(E + moff, MC)], dst_x, ms)
        pltpu.async_copy(et_hbm.at[pl.ds(moff, MC)], typ_x, ms)

    def _wait_meta(side):
        dst_x, typ_x, ms = metas[side]
        pltpu.make_async_copy(ei_hbm.at[pl.ds(E, MC)], dst_x, ms).wait()
        pltpu.make_async_copy(et_hbm.at[pl.ds(0, MC)], typ_x, ms).wait()

    def _chunk(side, j, p, skip_wait):
        dst_x, typ_x, _ = metas[side]
        cx, sx = cidxs[p], ssems[p]
        if not skip_wait:
            pltpu.make_async_copy(oh_v, c_sp.at[cx], sx).wait()
        for g in range(CC // 16):
            sl = pl.ds(g * 16, 16)
            t = typ_x[pl.ds(j * CC + g * 16, 16)]
            d = dst_x[pl.ds(j * CC + g * 16, 16)]
            cx[sl] = _fam_base(t, d, base_v, s_id, g)
        pltpu.async_copy(oh_v, c_sp.at[cx], sx, add=True)

    def _ubody(bi, side, poff, fire_next):
        _wait_meta(side)
        if fire_next:
            _fire_meta(bi + 1, 1 - side)
        for j in range(CPBC):
            _chunk(side, j, (poff + j) % 2, False)

    _fire_meta(0, 0)
    _wait_meta(0)
    _fire_meta(1, 1)
    for j in range(CPBC):
        _chunk(0, j, j % 2, j < 2)
    _ubody(1, 1, 1, True)

    @pl.loop(2, NBLKC - 1, step=2)
    def _steady(b):
        _ubody(b, 0, 0, True)
        _ubody(b + 1, 1, 1, True)

    _ubody(NBLKC - 1, 0, 0, False)
    pltpu.make_async_copy(oh_v, c_sp.at[cidx_a], ssem_a).wait()
    pltpu.make_async_copy(oh_v, c_sp.at[cidx_b], ssem_b).wait()
    plsc.subcore_barrier()

    pltpu.sync_copy(c_sp.at[pl.ds(s_id * AZ, AZ)],
                    c_out.at[c_id, pl.ds(s_id * AZ, AZ)])


def _acc_body(x_hbm, ei_hbm, et_hbm, cnt_hbm, a_out,
              src_a, src_b, dst_a, dst_b, typ_a, typ_b,
              fidx_0, fidx_1, fidx_2, rows_0, rows_1, rows_2, a_sp,
              msem_a, msem_b, gsem_0, gsem_1, gsem_2,
              ssem_0, ssem_1, ssem_2, zsem):
    del cnt_hbm
    c_id = lax.axis_index("c")
    s_id = lax.axis_index("s")
    base = c_id * H
    tbase = s_id * EPT
    base_v = jnp.broadcast_to(base, (16,))
    zf = jnp.zeros((16,), jnp.float32)

    def _zrow(i, _):
        for j in range(D // 16):
            rows_0[i, pl.ds(j * 16, 16)] = zf
            rows_1[i, pl.ds(j * 16, 16)] = zf
        return 0
    lax.fori_loop(0, CA, _zrow, 0)
    nz = AZ // (2 * CA)
    rz = AZ - nz * 2 * CA
    for j in range(nz):
        pltpu.async_copy(
            rows_0, a_sp.at[pl.ds(s_id * AZ + 2 * j * CA, CA)], zsem)
        pltpu.async_copy(
            rows_1, a_sp.at[pl.ds(s_id * AZ + (2 * j + 1) * CA, CA)], zsem)
    pltpu.async_copy(rows_0.at[pl.ds(0, rz)],
                     a_sp.at[pl.ds(s_id * AZ + nz * 2 * CA, rz)], zsem)
    for j in range(2 * nz):
        pltpu.make_async_copy(rows_0, a_sp.at[pl.ds(0, CA)], zsem).wait()
    pltpu.make_async_copy(rows_0.at[pl.ds(0, rz)],
                          a_sp.at[pl.ds(0, rz)], zsem).wait()
    plsc.subcore_barrier()

    metas = ((src_a, dst_a, typ_a, msem_a), (src_b, dst_b, typ_b, msem_b))
    fidxs = (fidx_0, fidx_1, fidx_2)
    rows = (rows_0, rows_1, rows_2)
    gsems = (gsem_0, gsem_1, gsem_2)
    ssems = (ssem_0, ssem_1, ssem_2)

    def _fire_meta(bi, side, ln):
        src_x, dst_x, typ_x, ms = metas[side]
        moff = tbase + bi * MA
        pltpu.async_copy(ei_hbm.at[pl.ds(moff, ln)], src_x.at[pl.ds(0, ln)], ms)
        pltpu.async_copy(ei_hbm.at[pl.ds(E + moff, ln)],
                         dst_x.at[pl.ds(0, ln)], ms)
        pltpu.async_copy(et_hbm.at[pl.ds(moff, ln)], typ_x.at[pl.ds(0, ln)], ms)

    def _wait_meta(side, ln):
        src_x, dst_x, typ_x, ms = metas[side]
        pltpu.make_async_copy(ei_hbm.at[pl.ds(0, ln)],
                              src_x.at[pl.ds(0, ln)], ms).wait()
        pltpu.make_async_copy(ei_hbm.at[pl.ds(0, ln)],
                              dst_x.at[pl.ds(0, ln)], ms).wait()
        pltpu.make_async_copy(et_hbm.at[pl.ds(0, ln)],
                              typ_x.at[pl.ds(0, ln)], ms).wait()

    def _chunk(side, j, skip_swait, skip_prev):
        src_x, dst_x, typ_x, _ = metas[side]
        p = j % 3
        r = (j + 1) % 3
        if not skip_swait:
            pltpu.make_async_copy(rows[p], a_sp.at[fidxs[p]], ssems[p]).wait()
        t = typ_x[pl.ds(j * CA, 16)]
        d = dst_x[pl.ds(j * CA, 16)]
        fidxs[p][pl.ds(0, 16)] = _fam_base(t, d, base_v, s_id, 0)
        pltpu.async_copy(x_hbm.at[src_x.at[pl.ds(j * CA, CA)]],
                         rows[p], gsems[p])
        if not skip_prev:
            if j >= 2:
                gsrc = x_hbm.at[src_x.at[pl.ds((j - 2) * CA, CA)]]
            else:
                src_y = metas[1 - side][0]
                gsrc = x_hbm.at[src_y.at[pl.ds((j - 2 + CPB) * CA, CA)]]
            pltpu.make_async_copy(gsrc, rows[r], gsems[r]).wait()
            pltpu.async_copy(rows[r], a_sp.at[fidxs[r]], ssems[r], add=True)

    def _ubody(bi, side, fire_mode):
        _wait_meta(side, MA)
        for j in range(CPB):
            _chunk(side, j, False, False)
            if j == 1 and fire_mode == 1:
                _fire_meta(bi + 1, 1 - side, MA)
            elif j == 1 and fire_mode == 2:
                _fire_meta(bi + 1, 1 - side, TAILE)

    _fire_meta(0, 0, MA)
    _fire_meta(1, 1, MA)
    _wait_meta(0, MA)
    for j in range(CPB):
        _chunk(0, j, j < 3, j < 2)
    _ubody(1, 1, 1)

    @pl.loop(2, NBLK - 1, step=2)
    def _steady(b):
        _ubody(b, 0, 1)
        _ubody(b + 1, 1, 1)

    _ubody(NBLK - 1, 0, 2)
    _wait_meta(1, TAILE)
    for j in range(TAILC):
        _chunk(1, j, False, False)
    src_x = metas[1][0]
    for j in (TAILC - 2, TAILC - 1):
        r = j % 3
        pltpu.make_async_copy(x_hbm.at[src_x.at[pl.ds(j * CA, CA)]],
                              rows[r], gsems[r]).wait()
        pltpu.async_copy(rows[r], a_sp.at[fidxs[r]], ssems[r], add=True)
    for r in range(3):
        pltpu.make_async_copy(rows[r], a_sp.at[fidxs[r]], ssems[r]).wait()
    plsc.subcore_barrier()

    pltpu.sync_copy(a_sp.at[pl.ds(s_id * AZ, AZ)],
                    a_out.at[c_id, pl.ds(s_id * AZ, AZ)])


def _make_sc_calls():
    mesh = plsc.VectorSubcoreMesh(core_axis_name="c", subcore_axis_name="s",
                                  num_cores=NC, num_subcores=NS)
    cnt_call = pl.kernel(
        _cnt_body,
        out_type=jax.ShapeDtypeStruct((NC, AROWS, D), jnp.float32),
        mesh=mesh,
        scratch_types=[
            pltpu.VMEM((MC,), jnp.int32),
            pltpu.VMEM((MC,), jnp.int32),
            pltpu.VMEM((MC,), jnp.int32),
            pltpu.VMEM((MC,), jnp.int32),
            pltpu.VMEM((CC,), jnp.int32),
            pltpu.VMEM((CC,), jnp.int32),
            pltpu.VMEM((CC, D), jnp.float32),
            pltpu.VMEM_SHARED((AROWS, D), jnp.float32),
            pltpu.SemaphoreType.DMA,
            pltpu.SemaphoreType.DMA,
            pltpu.SemaphoreType.DMA,
            pltpu.SemaphoreType.DMA,
            pltpu.SemaphoreType.DMA,
        ],
    )
    acc_call = pl.kernel(
        _acc_body,
        out_type=jax.ShapeDtypeStruct((NC, AROWS, D), jnp.float32),
        mesh=mesh,
        scratch_types=[
            pltpu.VMEM((MA,), jnp.int32),
            pltpu.VMEM((MA,), jnp.int32),
            pltpu.VMEM((MA,), jnp.int32),
            pltpu.VMEM((MA,), jnp.int32),
            pltpu.VMEM((MA,), jnp.int32),
            pltpu.VMEM((MA,), jnp.int32),
            pltpu.VMEM((CA,), jnp.int32),
            pltpu.VMEM((CA,), jnp.int32),
            pltpu.VMEM((CA,), jnp.int32),
            pltpu.VMEM((CA, D), jnp.float32),
            pltpu.VMEM((CA, D), jnp.float32),
            pltpu.VMEM((CA, D), jnp.float32),
            pltpu.VMEM_SHARED((AROWS, D), jnp.float32),
            pltpu.SemaphoreType.DMA,
            pltpu.SemaphoreType.DMA,
            pltpu.SemaphoreType.DMA,
            pltpu.SemaphoreType.DMA,
            pltpu.SemaphoreType.DMA,
            pltpu.SemaphoreType.DMA,
            pltpu.SemaphoreType.DMA,
            pltpu.SemaphoreType.DMA,
            pltpu.SemaphoreType.DMA,
        ],
    )
    return cnt_call, acc_call


_cnt_call, _acc_call = _make_sc_calls()


def _tc_body(a0_ref, a1_ref, a2_ref, c0_ref, c1_ref, c2_ref,
             wo_ref, wi_ref, ws_ref, bo_ref, bi_ref, bs_ref, out_ref):
    dn = (((1,), (1,)), ((), ()))
    m = lax.dot_general(a0_ref[0], wo_ref[...], dn,
                        preferred_element_type=jnp.float32,
                        precision=lax.Precision.HIGHEST)
    m += lax.dot_general(a1_ref[0], wi_ref[...], dn,
                         preferred_element_type=jnp.float32,
                         precision=lax.Precision.HIGHEST)
    m += lax.dot_general(a2_ref[0], ws_ref[...], dn,
                         preferred_element_type=jnp.float32,
                         precision=lax.Precision.HIGHEST)
    c0 = c0_ref[0][:, 0:1]
    c1 = c1_ref[0][:, 0:1]
    c2 = c2_ref[0][:, 0:1]
    m += c0 * bo_ref[...] + c1 * bi_ref[...] + c2 * bs_ref[...]
    inv = 1.0 / jnp.maximum(c0 + c1 + c2, 1.0)
    out_ref[...] = jnp.maximum(m * inv, 0.0)


_tc_call = pl.pallas_call(
    _tc_body,
    grid=(NC, NB),
    in_specs=[
        pl.BlockSpec((1, BN, D), lambda c, b: (c, 0 * NB + b, 0)),
        pl.BlockSpec((1, BN, D), lambda c, b: (c, 1 * NB + b, 0)),
        pl.BlockSpec((1, BN, D), lambda c, b: (c, 2 * NB + b, 0)),
        pl.BlockSpec((1, BN, D), lambda c, b: (c, 0 * NB + b, 0)),
        pl.BlockSpec((1, BN, D), lambda c, b: (c, 1 * NB + b, 0)),
        pl.BlockSpec((1, BN, D), lambda c, b: (c, 2 * NB + b, 0)),
        pl.BlockSpec((D, D), lambda c, b: (0, 0)),
        pl.BlockSpec((D, D), lambda c, b: (0, 0)),
        pl.BlockSpec((D, D), lambda c, b: (0, 0)),
        pl.BlockSpec((1, D), lambda c, b: (0, 0)),
        pl.BlockSpec((1, D), lambda c, b: (0, 0)),
        pl.BlockSpec((1, D), lambda c, b: (0, 0)),
    ],
    out_specs=pl.BlockSpec((BN, D), lambda c, b: (c * NB + b, 0)),
    out_shape=jax.ShapeDtypeStruct((N, D), jnp.float32),
)


@jax.jit
def kernel(x, edge_index, edge_types, W_o, b_o, W_i, b_i, W_s, b_s):
    ei = edge_index.reshape(2 * E)
    cnt = _cnt_call(ei, edge_types)
    a_acc = _acc_call(x, ei, edge_types, cnt)
    return _tc_call(a_acc, a_acc, a_acc, cnt, cnt, cnt, W_o, W_i, W_s,
                    b_o.reshape(1, D), b_i.reshape(1, D), b_s.reshape(1, D))
